# baseline (speedup 1.0000x reference)
import functools

import jax
import jax.numpy as jnp
from jax import lax
from jax.experimental import pallas as pl
from jax.experimental.pallas import tpu as pltpu
from jax.experimental.pallas import tpu_sc as plsc

VOC_SIZE = 1000000
EMB_SIZE = 16
BATCH = 16384

_NC = 2
_NS = 16
_NW = _NC * _NS
_BPW = BATCH // _NW
_NT = _BPW // 16


def _glove_body(center_hbm, outside_hbm, coocs_hbm, w_hbm,
                ce_hbm, oe_hbm, cb_hbm, ob_hbm, out_hbm,
                cidx_v, oidx_v, ce_v, oe_v, cb_v, ob_v, cooc_v, wv_v,
                out_v, sem):
    wid = lax.axis_index("s") * _NC + lax.axis_index("c")
    base = wid * _BPW

    pltpu.sync_copy(center_hbm.at[pl.ds(base, _BPW)], cidx_v)
    pltpu.sync_copy(outside_hbm.at[pl.ds(base, _BPW)], oidx_v)

    copies = [
        pltpu.async_copy(ce_hbm.at[cidx_v], ce_v, sem),
        pltpu.async_copy(oe_hbm.at[oidx_v], oe_v, sem),
        pltpu.async_copy(cb_hbm.at[cidx_v], cb_v, sem),
        pltpu.async_copy(ob_hbm.at[oidx_v], ob_v, sem),
        pltpu.async_copy(coocs_hbm.at[pl.ds(base, _BPW)], cooc_v, sem),
        pltpu.async_copy(w_hbm.at[pl.ds(base, _BPW)], wv_v, sem),
    ]
    for c in copies:
        c.wait()

    lane = lax.broadcasted_iota(jnp.int32, (16,), 0)

    def tile_body(t, loss_acc):
        row = t * 16 + lane
        ip = jnp.zeros((16,), jnp.float32)
        for j in range(EMB_SIZE):
            col = jnp.full((16,), j, jnp.int32)
            cej = plsc.load_gather(ce_v, [row, col])
            oej = plsc.load_gather(oe_v, [row, col])
            ip = ip + cej * oej
        cb = plsc.load_gather(cb_v, [row])
        tb = plsc.load_gather(ob_v, [row])
        cooc = plsc.load_gather(cooc_v, [row])
        w = plsc.load_gather(wv_v, [row])
        r = ip + cb + tb - cooc
        return loss_acc + w * r * r

    loss = lax.fori_loop(0, _NT, tile_body, jnp.zeros((16,), jnp.float32))
    out_v[...] = loss
    pltpu.sync_copy(out_v, out_hbm.at[wid])


@functools.partial(jax.jit, static_argnums=())
def _glove_partials(center, outside, coocs, weighting, ce, oe, cb, ob):
    mesh = plsc.VectorSubcoreMesh(core_axis_name="c", subcore_axis_name="s")
    k = functools.partial(
        pl.kernel,
        mesh=mesh,
        out_type=jax.ShapeDtypeStruct((_NW, 16), jnp.float32),
        scratch_types=[
            pltpu.VMEM((_BPW,), jnp.int32),
            pltpu.VMEM((_BPW,), jnp.int32),
            pltpu.VMEM((_BPW, EMB_SIZE), jnp.float32),
            pltpu.VMEM((_BPW, EMB_SIZE), jnp.float32),
            pltpu.VMEM((_BPW,), jnp.float32),
            pltpu.VMEM((_BPW,), jnp.float32),
            pltpu.VMEM((_BPW,), jnp.float32),
            pltpu.VMEM((_BPW,), jnp.float32),
            pltpu.VMEM((16,), jnp.float32),
            pltpu.SemaphoreType.DMA,
        ],
        compiler_params=pltpu.CompilerParams(
            needs_layout_passes=False,
            use_tc_tiling_on_sc=False,
        ),
    )(_glove_body)
    return k(center, outside, coocs, weighting, ce, oe, cb, ob)


def kernel(center, outside, coocs, weighting, center_embedding,
           outside_embedding, center_bias, outside_bias):
    parts = _glove_partials(
        center.reshape(-1), outside.reshape(-1),
        coocs.reshape(-1), weighting.reshape(-1),
        center_embedding, outside_embedding,
        center_bias.reshape(-1), outside_bias.reshape(-1),
    )
    return jnp.sum(parts)


# speedup vs baseline: 3.5265x; 3.5265x over previous
import functools

import jax
import jax.numpy as jnp
from jax import lax
from jax.experimental import pallas as pl
from jax.experimental.pallas import tpu as pltpu
from jax.experimental.pallas import tpu_sc as plsc

VOC_SIZE = 1000000
EMB_SIZE = 16
BATCH = 16384

_NC = 2
_NS = 16
_NW = _NC * _NS
_BPW = BATCH // _NW
_G = 8
_NG = _BPW // _G
_WMAX = VOC_SIZE - 128


def _win_base(v):
    c = lax.shift_left(lax.shift_right_logical(v, 7), 7)
    return pl.multiple_of(jnp.minimum(c, _WMAX), 128)


def _glove_body(center_hbm, outside_hbm, coocs_hbm, w_hbm,
                ceT_hbm, oeT_hbm, cb_hbm, ob_hbm, out_hbm,
                cidx_v, oidx_v, cooc_v, wv_v,
                cewA, cewB, oewA, oewB, cbwA, cbwB, obwA, obwB,
                out_v, semA, semB):
    wid = lax.axis_index("s") * _NC + lax.axis_index("c")
    base = wid * _BPW

    pltpu.sync_copy(center_hbm.at[pl.ds(base, _BPW)], cidx_v)
    pltpu.sync_copy(outside_hbm.at[pl.ds(base, _BPW)], oidx_v)
    pltpu.sync_copy(coocs_hbm.at[pl.ds(base, _BPW)], cooc_v)
    pltpu.sync_copy(w_hbm.at[pl.ds(base, _BPW)], wv_v)

    lane = lax.broadcasted_iota(jnp.int32, (16,), 0)

    def fire(vc, vo, l0, cew, oew, cbw, obw, sem):
        for j in range(_G):
            v = vc[l0 + j]
            u = vo[l0 + j]
            cv = _win_base(v)
            cu = _win_base(u)
            pltpu.make_async_copy(
                ceT_hbm.at[:, pl.ds(cv, 128)],
                cew.at[pl.ds(j * 16, 16), :], sem).start()
            pltpu.make_async_copy(
                oeT_hbm.at[:, pl.ds(cu, 128)],
                oew.at[pl.ds(j * 16, 16), :], sem).start()
            pltpu.make_async_copy(cb_hbm.at[pl.ds(cv, 128)],
                                  cbw.at[j], sem).start()
            pltpu.make_async_copy(ob_hbm.at[pl.ds(cu, 128)],
                                  obw.at[j], sem).start()

    def drain(cew, oew, cbw, obw, sem):
        for j in range(_G):
            pltpu.make_async_copy(ceT_hbm.at[:, pl.ds(0, 128)],
                                  cew.at[pl.ds(j * 16, 16), :], sem).wait()
            pltpu.make_async_copy(oeT_hbm.at[:, pl.ds(0, 128)],
                                  oew.at[pl.ds(j * 16, 16), :], sem).wait()
            pltpu.make_async_copy(cb_hbm.at[pl.ds(0, 128)],
                                  cbw.at[j], sem).wait()
            pltpu.make_async_copy(ob_hbm.at[pl.ds(0, 128)],
                                  obw.at[j], sem).wait()

    def compute(vc, vo, cvec, wvec, l0, cew, oew, cbw, obw, acc):
        for j in range(_G):
            v = vc[l0 + j]
            u = vo[l0 + j]
            lv = v - jnp.minimum(
                lax.shift_left(lax.shift_right_logical(v, 7), 7), _WMAX)
            lu = u - jnp.minimum(
                lax.shift_left(lax.shift_right_logical(u, 7), 7), _WMAX)
            lv16 = jnp.broadcast_to(lv, (16,))
            lu16 = jnp.broadcast_to(lu, (16,))
            ce = plsc.load_gather(cew, [j * 16 + lane, lv16])
            oe = plsc.load_gather(oew, [j * 16 + lane, lu16])
            ip = jnp.sum(ce * oe)
            cb = plsc.load_gather(cbw, [jnp.full((16,), j, jnp.int32), lv16])[0]
            tb = plsc.load_gather(obw, [jnp.full((16,), j, jnp.int32), lu16])[0]
            r = ip + cb + tb - cvec[l0 + j]
            acc = acc + wvec[l0 + j] * r * r
        return acc

    vc0 = cidx_v[pl.ds(0, 16)]
    vo0 = oidx_v[pl.ds(0, 16)]
    fire(vc0, vo0, 0, cewA, oewA, cbwA, obwA, semA)
    fire(vc0, vo0, 8, cewB, oewB, cbwB, obwB, semB)

    def pair_body(p, carry):
        vc, vo, acc = carry
        pnext = jnp.minimum(p + 1, _NG // 2 - 1) * 16
        vcn = cidx_v[pl.ds(pnext, 16)]
        von = oidx_v[pl.ds(pnext, 16)]
        cvec = cooc_v[pl.ds(p * 16, 16)]
        wvec = wv_v[pl.ds(p * 16, 16)]

        drain(cewA, oewA, cbwA, obwA, semA)
        acc = compute(vc, vo, cvec, wvec, 0, cewA, oewA, cbwA, obwA, acc)

        @pl.when(p < _NG // 2 - 1)
        def _():
            fire(vcn, von, 0, cewA, oewA, cbwA, obwA, semA)

        drain(cewB, oewB, cbwB, obwB, semB)
        acc = compute(vc, vo, cvec, wvec, 8, cewB, oewB, cbwB, obwB, acc)

        @pl.when(p < _NG // 2 - 1)
        def _():
            fire(vcn, von, 8, cewB, oewB, cbwB, obwB, semB)

        return (vcn, von, acc)

    _, _, acc = lax.fori_loop(
        0, _NG // 2, pair_body, (vc0, vo0, jnp.float32(0.0)))

    out_v[...] = jnp.where(lane == 0, acc, 0.0)
    pltpu.sync_copy(out_v, out_hbm.at[wid])


def _glove_partials(center, outside, coocs, weighting, ceT, oeT, cb, ob):
    mesh = plsc.VectorSubcoreMesh(core_axis_name="c", subcore_axis_name="s")
    k = functools.partial(
        pl.kernel,
        mesh=mesh,
        out_type=jax.ShapeDtypeStruct((_NW, 16), jnp.float32),
        scratch_types=[
            pltpu.VMEM((_BPW,), jnp.int32),
            pltpu.VMEM((_BPW,), jnp.int32),
            pltpu.VMEM((_BPW,), jnp.float32),
            pltpu.VMEM((_BPW,), jnp.float32),
            pltpu.VMEM((_G * 16, 128), jnp.float32),
            pltpu.VMEM((_G * 16, 128), jnp.float32),
            pltpu.VMEM((_G * 16, 128), jnp.float32),
            pltpu.VMEM((_G * 16, 128), jnp.float32),
            pltpu.VMEM((_G, 128), jnp.float32),
            pltpu.VMEM((_G, 128), jnp.float32),
            pltpu.VMEM((_G, 128), jnp.float32),
            pltpu.VMEM((_G, 128), jnp.float32),
            pltpu.VMEM((16,), jnp.float32),
            pltpu.SemaphoreType.DMA,
            pltpu.SemaphoreType.DMA,
        ],
        compiler_params=pltpu.CompilerParams(
            needs_layout_passes=False,
            use_tc_tiling_on_sc=True,
        ),
    )(_glove_body)
    return k(center, outside, coocs, weighting, ceT, oeT, cb, ob)


def kernel(center, outside, coocs, weighting, center_embedding,
           outside_embedding, center_bias, outside_bias):
    parts = _glove_partials(
        center.reshape(-1), outside.reshape(-1),
        coocs.reshape(-1), weighting.reshape(-1),
        center_embedding.T, outside_embedding.T,
        center_bias.reshape(-1), outside_bias.reshape(-1),
    )
    return jnp.sum(parts)


# speedup vs baseline: 5.5852x; 1.5838x over previous
import functools

import jax
import jax.numpy as jnp
from jax import lax
from jax.experimental import pallas as pl
from jax.experimental.pallas import tpu as pltpu
from jax.experimental.pallas import tpu_sc as plsc

VOC_SIZE = 1000000
EMB_SIZE = 16
BATCH = 16384

_NC = 2
_NS = 16
_NW = _NC * _NS
_BPW = BATCH // _NW
_G = 8
_NG = _BPW // _G
_GW = _G * 128
_WMAX = VOC_SIZE - 128


def _win_base(v):
    c = lax.shift_left(lax.shift_right_logical(v, 7), 7)
    return pl.multiple_of(jnp.minimum(c, _WMAX), 128)


def _glove_body(center_hbm, outside_hbm, coocs_hbm, w_hbm,
                ceT_hbm, oeT_hbm, cb_hbm, ob_hbm, out_hbm,
                cidx_v, oidx_v, cooc_v, wv_v,
                cewA, cewB, oewA, oewB, cbwA, cbwB, obwA, obwB,
                out_v, semA, semB):
    wid = lax.axis_index("s") * _NC + lax.axis_index("c")
    base = wid * _BPW

    pltpu.sync_copy(center_hbm.at[pl.ds(base, _BPW)], cidx_v)
    pltpu.sync_copy(outside_hbm.at[pl.ds(base, _BPW)], oidx_v)
    pltpu.sync_copy(coocs_hbm.at[pl.ds(base, _BPW)], cooc_v)
    pltpu.sync_copy(w_hbm.at[pl.ds(base, _BPW)], wv_v)

    lane = lax.broadcasted_iota(jnp.int32, (16,), 0)
    zeros16 = jnp.zeros((16,), jnp.int32)

    def fire(vc, vo, l0, cew, oew, cbw, obw, sem):
        for j in range(_G):
            v = vc[l0 + j]
            u = vo[l0 + j]
            cv = _win_base(v)
            cu = _win_base(u)
            pltpu.make_async_copy(
                ceT_hbm.at[:, pl.ds(cv, 128)],
                cew.at[:, pl.ds(j * 128, 128)], sem).start()
            pltpu.make_async_copy(
                oeT_hbm.at[:, pl.ds(cu, 128)],
                oew.at[:, pl.ds(j * 128, 128)], sem).start()
            pltpu.make_async_copy(cb_hbm.at[:, pl.ds(cv, 128)],
                                  cbw.at[pl.ds(j, 1), :], sem).start()
            pltpu.make_async_copy(ob_hbm.at[:, pl.ds(cu, 128)],
                                  obw.at[pl.ds(j, 1), :], sem).start()

    def drain(cew, oew, cbw, obw, sem):
        pltpu.make_async_copy(ceT_hbm.at[:, pl.ds(0, _GW)], cew, sem).wait()
        pltpu.make_async_copy(oeT_hbm.at[:, pl.ds(0, _GW)], oew, sem).wait()
        for j in range(_G):
            pltpu.make_async_copy(cb_hbm.at[:, pl.ds(0, 128)],
                                  cbw.at[pl.ds(j, 1), :], sem).wait()
            pltpu.make_async_copy(ob_hbm.at[:, pl.ds(0, 128)],
                                  obw.at[pl.ds(j, 1), :], sem).wait()

    def compute(vc, vo, cvec, wvec, l0, cew, oew, cbw, obw, acc):
        for j in range(_G):
            v = vc[l0 + j]
            u = vo[l0 + j]
            lv = v - jnp.minimum(
                lax.shift_left(lax.shift_right_logical(v, 7), 7), _WMAX)
            lu = u - jnp.minimum(
                lax.shift_left(lax.shift_right_logical(u, 7), 7), _WMAX)
            lv16 = jnp.broadcast_to(lv, (16,))
            lu16 = jnp.broadcast_to(lu, (16,))
            j16 = jnp.full((16,), j, jnp.int32)
            ce = plsc.load_gather(cew, [lane, j * 128 + lv16])
            oe = plsc.load_gather(oew, [lane, j * 128 + lu16])
            ip = jnp.sum(ce * oe)
            cb = plsc.load_gather(cbw, [j16, lv16])[0]
            tb = plsc.load_gather(obw, [j16, lu16])[0]
            r = ip + cb + tb - cvec[l0 + j]
            acc = acc + wvec[l0 + j] * r * r
        return acc

    vc0 = cidx_v[pl.ds(0, 16)]
    vo0 = oidx_v[pl.ds(0, 16)]
    fire(vc0, vo0, 0, cewA, oewA, cbwA, obwA, semA)
    fire(vc0, vo0, 8, cewB, oewB, cbwB, obwB, semB)

    def pair_body(p, carry):
        vc, vo, acc = carry
        pnext = jnp.minimum(p + 1, _NG // 2 - 1) * 16
        vcn = cidx_v[pl.ds(pnext, 16)]
        von = oidx_v[pl.ds(pnext, 16)]
        cvec = cooc_v[pl.ds(p * 16, 16)]
        wvec = wv_v[pl.ds(p * 16, 16)]

        drain(cewA, oewA, cbwA, obwA, semA)
        acc = compute(vc, vo, cvec, wvec, 0, cewA, oewA, cbwA, obwA, acc)

        @pl.when(p < _NG // 2 - 1)
        def _():
            fire(vcn, von, 0, cewA, oewA, cbwA, obwA, semA)

        drain(cewB, oewB, cbwB, obwB, semB)
        acc = compute(vc, vo, cvec, wvec, 8, cewB, oewB, cbwB, obwB, acc)

        @pl.when(p < _NG // 2 - 1)
        def _():
            fire(vcn, von, 8, cewB, oewB, cbwB, obwB, semB)

        return (vcn, von, acc)

    _, _, acc = lax.fori_loop(
        0, _NG // 2, pair_body, (vc0, vo0, jnp.float32(0.0)))

    out_v[...] = jnp.where(lane == 0, acc, 0.0)
    pltpu.sync_copy(out_v, out_hbm.at[wid])


def _glove_partials(center, outside, coocs, weighting, ceT, oeT, cb, ob):
    mesh = plsc.VectorSubcoreMesh(core_axis_name="c", subcore_axis_name="s")
    k = functools.partial(
        pl.kernel,
        mesh=mesh,
        out_type=jax.ShapeDtypeStruct((_NW, 16), jnp.float32),
        scratch_types=[
            pltpu.VMEM((_BPW,), jnp.int32),
            pltpu.VMEM((_BPW,), jnp.int32),
            pltpu.VMEM((_BPW,), jnp.float32),
            pltpu.VMEM((_BPW,), jnp.float32),
            pltpu.VMEM((EMB_SIZE, _GW), jnp.float32),
            pltpu.VMEM((EMB_SIZE, _GW), jnp.float32),
            pltpu.VMEM((EMB_SIZE, _GW), jnp.float32),
            pltpu.VMEM((EMB_SIZE, _GW), jnp.float32),
            pltpu.VMEM((_G, 128), jnp.float32),
            pltpu.VMEM((_G, 128), jnp.float32),
            pltpu.VMEM((_G, 128), jnp.float32),
            pltpu.VMEM((_G, 128), jnp.float32),
            pltpu.VMEM((16,), jnp.float32),
            pltpu.SemaphoreType.DMA,
            pltpu.SemaphoreType.DMA,
        ],
        compiler_params=pltpu.CompilerParams(
            needs_layout_passes=False,
            use_tc_tiling_on_sc=True,
        ),
    )(_glove_body)
    return k(center, outside, coocs, weighting, ceT, oeT, cb, ob)


def kernel(center, outside, coocs, weighting, center_embedding,
           outside_embedding, center_bias, outside_bias):
    parts = _glove_partials(
        center.reshape(-1), outside.reshape(-1),
        coocs.reshape(-1), weighting.reshape(-1),
        center_embedding.T, outside_embedding.T,
        center_bias.T, outside_bias.T,
    )
    return jnp.sum(parts)


# speedup vs baseline: 5.8421x; 1.0460x over previous
import functools

import jax
import jax.numpy as jnp
from jax import lax
from jax.experimental import pallas as pl
from jax.experimental.pallas import tpu as pltpu
from jax.experimental.pallas import tpu_sc as plsc

VOC_SIZE = 1000000
EMB_SIZE = 16
BATCH = 16384

_NC = 2
_NS = 16
_NW = _NC * _NS
_BPW = BATCH // _NW
_G = 8
_NG = _BPW // _G
_GW = _G * 128
_WMAX = VOC_SIZE - 128


def _win_base_vec(v):
    c = lax.shift_left(lax.shift_right_logical(v, 7), 7)
    return jnp.minimum(c, _WMAX)


def _glove_body(center_hbm, outside_hbm, coocs_hbm, w_hbm,
                ceT_hbm, oeT_hbm, cb_hbm, ob_hbm, out_hbm,
                cidx_v, oidx_v, cooc_v, wv_v,
                cewA, cewB, oewA, oewB, cbwA, cbwB, obwA, obwB,
                out_v, semA, semB):
    wid = lax.axis_index("s") * _NC + lax.axis_index("c")
    base = wid * _BPW

    pltpu.sync_copy(center_hbm.at[pl.ds(base, _BPW)], cidx_v)
    pltpu.sync_copy(outside_hbm.at[pl.ds(base, _BPW)], oidx_v)
    pltpu.sync_copy(coocs_hbm.at[pl.ds(base, _BPW)], cooc_v)
    pltpu.sync_copy(w_hbm.at[pl.ds(base, _BPW)], wv_v)

    lane = lax.broadcasted_iota(jnp.int32, (16,), 0)
    zeros16 = jnp.zeros((16,), jnp.int32)

    def fire(cvv, cuv, l0, cew, oew, cbw, obw, sem):
        for j in range(_G):
            cv = pl.multiple_of(cvv[l0 + j], 128)
            cu = pl.multiple_of(cuv[l0 + j], 128)
            pltpu.make_async_copy(
                ceT_hbm.at[:, pl.ds(cv, 128)],
                cew.at[:, pl.ds(j * 128, 128)], sem).start()
            pltpu.make_async_copy(
                oeT_hbm.at[:, pl.ds(cu, 128)],
                oew.at[:, pl.ds(j * 128, 128)], sem).start()
            pltpu.make_async_copy(cb_hbm.at[:, pl.ds(cv, 128)],
                                  cbw.at[pl.ds(j, 1), :], sem).start()
            pltpu.make_async_copy(ob_hbm.at[:, pl.ds(cu, 128)],
                                  obw.at[pl.ds(j, 1), :], sem).start()

    def drain(cew, oew, cbw, obw, sem):
        pltpu.make_async_copy(ceT_hbm.at[:, pl.ds(0, _GW)], cew, sem).wait()
        pltpu.make_async_copy(oeT_hbm.at[:, pl.ds(0, _GW)], oew, sem).wait()
        pltpu.make_async_copy(ceT_hbm.at[pl.ds(0, _G), pl.ds(0, 128)],
                              cbw, sem).wait()
        pltpu.make_async_copy(ceT_hbm.at[pl.ds(0, _G), pl.ds(0, 128)],
                              obw, sem).wait()

    def compute(lvv, luv, cvec, wvec, l0, cew, oew, cbw, obw, acc):
        for j in range(_G):
            lv16 = jnp.broadcast_to(lvv[l0 + j], (16,))
            lu16 = jnp.broadcast_to(luv[l0 + j], (16,))
            j16 = jnp.full((16,), j, jnp.int32)
            ce = plsc.load_gather(cew, [lane, j * 128 + lv16])
            oe = plsc.load_gather(oew, [lane, j * 128 + lu16])
            ip = jnp.sum(ce * oe)
            cb = plsc.load_gather(cbw, [j16, lv16])[0]
            tb = plsc.load_gather(obw, [j16, lu16])[0]
            r = ip + cb + tb - cvec[l0 + j]
            acc = acc + wvec[l0 + j] * r * r
        return acc

    vc0 = cidx_v[pl.ds(0, 16)]
    vo0 = oidx_v[pl.ds(0, 16)]
    cv0 = _win_base_vec(vc0)
    cu0 = _win_base_vec(vo0)
    fire(cv0, cu0, 0, cewA, oewA, cbwA, obwA, semA)
    fire(cv0, cu0, 8, cewB, oewB, cbwB, obwB, semB)

    def pair_body(p, carry):
        vc, vo, cv, cu, acc = carry
        pnext = jnp.minimum(p + 1, _NG // 2 - 1) * 16
        vcn = cidx_v[pl.ds(pnext, 16)]
        von = oidx_v[pl.ds(pnext, 16)]
        cvn = _win_base_vec(vcn)
        cun = _win_base_vec(von)
        lvv = vc - cv
        luv = vo - cu
        cvec = cooc_v[pl.ds(p * 16, 16)]
        wvec = wv_v[pl.ds(p * 16, 16)]

        drain(cewA, oewA, cbwA, obwA, semA)
        acc = compute(lvv, luv, cvec, wvec, 0, cewA, oewA, cbwA, obwA, acc)

        @pl.when(p < _NG // 2 - 1)
        def _():
            fire(cvn, cun, 0, cewA, oewA, cbwA, obwA, semA)

        drain(cewB, oewB, cbwB, obwB, semB)
        acc = compute(lvv, luv, cvec, wvec, 8, cewB, oewB, cbwB, obwB, acc)

        @pl.when(p < _NG // 2 - 1)
        def _():
            fire(cvn, cun, 8, cewB, oewB, cbwB, obwB, semB)

        return (vcn, von, cvn, cun, acc)

    _, _, _, _, acc = lax.fori_loop(
        0, _NG // 2, pair_body, (vc0, vo0, cv0, cu0, jnp.float32(0.0)))

    out_v[...] = jnp.where(lane == 0, acc, 0.0)
    pltpu.sync_copy(out_v, out_hbm.at[wid])


def _glove_partials(center, outside, coocs, weighting, ceT, oeT, cb, ob):
    mesh = plsc.VectorSubcoreMesh(core_axis_name="c", subcore_axis_name="s")
    k = functools.partial(
        pl.kernel,
        mesh=mesh,
        out_type=jax.ShapeDtypeStruct((_NW, 16), jnp.float32),
        scratch_types=[
            pltpu.VMEM((_BPW,), jnp.int32),
            pltpu.VMEM((_BPW,), jnp.int32),
            pltpu.VMEM((_BPW,), jnp.float32),
            pltpu.VMEM((_BPW,), jnp.float32),
            pltpu.VMEM((EMB_SIZE, _GW), jnp.float32),
            pltpu.VMEM((EMB_SIZE, _GW), jnp.float32),
            pltpu.VMEM((EMB_SIZE, _GW), jnp.float32),
            pltpu.VMEM((EMB_SIZE, _GW), jnp.float32),
            pltpu.VMEM((_G, 128), jnp.float32),
            pltpu.VMEM((_G, 128), jnp.float32),
            pltpu.VMEM((_G, 128), jnp.float32),
            pltpu.VMEM((_G, 128), jnp.float32),
            pltpu.VMEM((16,), jnp.float32),
            pltpu.SemaphoreType.DMA,
            pltpu.SemaphoreType.DMA,
        ],
        compiler_params=pltpu.CompilerParams(
            needs_layout_passes=False,
            use_tc_tiling_on_sc=True,
        ),
    )(_glove_body)
    return k(center, outside, coocs, weighting, ceT, oeT, cb, ob)


def kernel(center, outside, coocs, weighting, center_embedding,
           outside_embedding, center_bias, outside_bias):
    parts = _glove_partials(
        center.reshape(-1), outside.reshape(-1),
        coocs.reshape(-1), weighting.reshape(-1),
        center_embedding.T, outside_embedding.T,
        center_bias.T, outside_bias.T,
    )
    return jnp.sum(parts)
